# baseline (speedup 1.0000x reference)
import functools

import jax
import jax.numpy as jnp
from jax import lax
from jax.experimental import pallas as pl
from jax.experimental.pallas import tpu as pltpu
from jax.experimental.pallas import tpu_sc as plsc

N_NODES = 10000
D = 128
N_EDGES = 320000

CHUNK = 128
CPW = 80
N_CHUNKS = 32 * CPW
E_PAD = N_CHUNKS * CHUNK
BLK = 8
N_BLK = CPW // BLK
N_PAD = 10240
ROWS_PER_TILE = N_PAD // 16

_mesh = plsc.VectorSubcoreMesh(core_axis_name="c", subcore_axis_name="s")
_f32 = jnp.float32
_sc_params = pltpu.CompilerParams(use_tc_tiling_on_sc=False)


@functools.partial(
    pl.kernel,
    out_type=jax.ShapeDtypeStruct((2, N_PAD, D), _f32),
    mesh=_mesh,
    scratch_types=[
        pltpu.VMEM((CHUNK,), jnp.int32),
        pltpu.VMEM((CHUNK, D), _f32),
        pltpu.VMEM((CHUNK, D), _f32),
        pltpu.VMEM_SHARED((N_PAD, D), _f32),
        pltpu.SemaphoreType.DMA,
    ],
    compiler_params=_sc_params,
)
def _deg_kernel(col_hbm, ones_hbm, zeros_hbm, deg_out, idx_v, ones_v, zero_v,
                acc_sh, sem):
    cid = lax.axis_index("c")
    sid = lax.axis_index("s")
    base = sid * ROWS_PER_TILE
    g0 = (sid * 2 + cid) * CPW

    pltpu.sync_copy(ones_hbm, ones_v)
    pltpu.sync_copy(zeros_hbm, zero_v)
    for k in range(ROWS_PER_TILE // CHUNK):
        pltpu.sync_copy(zero_v, acc_sh.at[pl.ds(base + k * CHUNK, CHUNK)])
    plsc.subcore_barrier()

    def body(j, carry):
        pltpu.sync_copy(col_hbm.at[g0 + j], idx_v)
        pltpu.sync_copy(ones_v, acc_sh.at[idx_v], add=True)
        return carry

    lax.fori_loop(0, CPW, body, 0)
    plsc.subcore_barrier()

    for k in range(ROWS_PER_TILE // CHUNK):
        pltpu.sync_copy(acc_sh.at[pl.ds(base + k * CHUNK, CHUNK)], zero_v)
        pltpu.sync_copy(zero_v, deg_out.at[cid, pl.ds(base + k * CHUNK, CHUNK)])


@functools.partial(
    pl.kernel,
    out_type=jax.ShapeDtypeStruct((2, N_PAD, D), _f32),
    mesh=_mesh,
    scratch_types=[
        pltpu.VMEM((4 * BLK, CHUNK), jnp.int32),
        pltpu.VMEM((CHUNK, D), _f32),
        pltpu.VMEM((CHUNK, D), _f32),
        pltpu.VMEM_SHARED((N_PAD, D), _f32),
        pltpu.SemaphoreType.DMA,
        pltpu.SemaphoreType.DMA,
        pltpu.SemaphoreType.DMA,
    ],
    compiler_params=_sc_params,
)
def _spmm_kernel(y_hbm, idx_hbm, zeros_hbm, s_out, idxp_v, gb_a, gb_b, acc_sh,
                 sem_a, sem_b, sem_i):
    cid = lax.axis_index("c")
    sid = lax.axis_index("s")
    base = sid * ROWS_PER_TILE
    r0_hbm = 2 * (sid * 2 + cid) * CPW
    gbs = (gb_a, gb_b)
    sems = (sem_a, sem_b)

    pltpu.sync_copy(zeros_hbm, gb_a)
    for k in range(ROWS_PER_TILE // CHUNK):
        pltpu.sync_copy(gb_a, acc_sh.at[pl.ds(base + k * CHUNK, CHUNK)])
    plsc.subcore_barrier()

    pltpu.sync_copy(idx_hbm.at[pl.ds(r0_hbm, 2 * BLK)],
                    idxp_v.at[pl.ds(0, 2 * BLK)])
    pltpu.async_copy(idx_hbm.at[pl.ds(r0_hbm + 2 * BLK, 2 * BLK)],
                     idxp_v.at[pl.ds(2 * BLK, 2 * BLK)], sem_i)
    pltpu.async_copy(y_hbm.at[idxp_v.at[0]], gb_a, sem_a)

    def block(b, carry):
        r0 = (b % 2) * (2 * BLK)
        rn = ((b + 1) % 2) * (2 * BLK)
        for i in range(BLK):
            cur, nxt = gbs[i % 2], gbs[(i + 1) % 2]
            scur, snxt = sems[i % 2], sems[(i + 1) % 2]
            if i < BLK - 1:
                pltpu.async_copy(y_hbm.at[idxp_v.at[r0 + 2 * i + 2]], nxt,
                                 snxt)
            else:
                @pl.when(b < N_BLK - 1)
                def _():
                    pltpu.make_async_copy(
                        idx_hbm.at[pl.ds(r0_hbm + (b + 1) * 2 * BLK, 2 * BLK)],
                        idxp_v.at[pl.ds(rn, 2 * BLK)], sem_i).wait()
                    pltpu.async_copy(y_hbm.at[idxp_v.at[rn]], nxt, snxt)
            pltpu.make_async_copy(y_hbm.at[idxp_v.at[r0 + 2 * i]], cur,
                                  scur).wait()
            pltpu.sync_copy(cur, acc_sh.at[idxp_v.at[r0 + 2 * i + 1]],
                            add=True)
            if i == BLK - 1:
                @pl.when(b < N_BLK - 2)
                def _():
                    pltpu.async_copy(
                        idx_hbm.at[pl.ds(r0_hbm + (b + 2) * 2 * BLK, 2 * BLK)],
                        idxp_v.at[pl.ds(r0, 2 * BLK)], sem_i)
        return carry

    lax.fori_loop(0, N_BLK, block, 0)
    plsc.subcore_barrier()

    for k in range(ROWS_PER_TILE // CHUNK):
        pltpu.sync_copy(acc_sh.at[pl.ds(base + k * CHUNK, CHUNK)], gb_a)
        pltpu.sync_copy(gb_a, s_out.at[cid, pl.ds(base + k * CHUNK, CHUNK)])


def _tc_prep_body(deg_ref, x_ref, w1_ref, d_ref, y_ref, dw_ref):
    d = lax.rsqrt(deg_ref[...] + 1.0)
    d_ref[...] = d
    xl = jnp.dot(x_ref[...], w1_ref[...], preferred_element_type=_f32)
    zpad = jnp.zeros((N_PAD - N_NODES, D), _f32)
    y_ref[0:N_NODES, :] = d[0:N_NODES] * xl
    y_ref[N_NODES:N_PAD, :] = zpad
    dw_ref[...] = jnp.broadcast_to(d, (N_PAD, D))


def _tc_mid_body(s_ref, y_ref, d_ref, b_ref, w_ref, out_ref):
    d = d_ref[0:N_NODES]
    s = s_ref[0, 0:N_NODES, :] + s_ref[1, 0:N_NODES, :] + y_ref[0:N_NODES, :]
    h = jnp.maximum(d * s + b_ref[...], 0.0)
    xl = jnp.dot(h, w_ref[...], preferred_element_type=_f32)
    out_ref[0:N_NODES, :] = d * xl
    out_ref[N_NODES:N_PAD, :] = jnp.zeros((N_PAD - N_NODES, D), _f32)


def _tc_final_body(s_ref, y_ref, d_ref, b_ref, c_ref, wo_ref, bo_ref,
                   out_ref):
    d = d_ref[0:N_NODES]
    s = s_ref[0, 0:N_NODES, :] + s_ref[1, 0:N_NODES, :] + y_ref[0:N_NODES, :]
    h2 = jnp.maximum(d * s + b_ref[...], 0.0)
    w = d * (c_ref[0:N_NODES] + d)
    z = jnp.sum(h2 * w, axis=0, keepdims=True)
    out_ref[...] = (jnp.dot(z, wo_ref[...], preferred_element_type=_f32)
                    * (1.0 / N_NODES) + bo_ref[...])


def kernel(x, edge_index, W1, b1, Wh, bh, Wo, bo):
    ei = edge_index.astype(jnp.int32)
    pad = jnp.full((E_PAD - N_EDGES,), N_NODES, jnp.int32)
    row2 = jnp.concatenate([ei[0], pad]).reshape(N_CHUNKS, CHUNK)
    col2 = jnp.concatenate([ei[1], pad]).reshape(N_CHUNKS, CHUNK)
    idx_fwd = jnp.stack([row2, col2], axis=1).reshape(2 * N_CHUNKS, CHUNK)
    idx_rev = jnp.stack([col2, row2], axis=1).reshape(2 * N_CHUNKS, CHUNK)

    ones_t = jnp.ones((CHUNK, D), _f32)
    zeros_t = jnp.zeros((CHUNK, D), _f32)

    degw = _deg_kernel(col2, ones_t, zeros_t)
    deg = degw[0, :, 0:1] + degw[1, :, 0:1]

    d_arr, y1, dwide = pl.pallas_call(
        _tc_prep_body,
        out_shape=(jax.ShapeDtypeStruct((N_PAD, 1), _f32),
                   jax.ShapeDtypeStruct((N_PAD, D), _f32),
                   jax.ShapeDtypeStruct((N_PAD, D), _f32)),
    )(deg, x, W1)

    s1 = _spmm_kernel(y1, idx_fwd, zeros_t)
    cw = _spmm_kernel(dwide, idx_rev, zeros_t)
    c_arr = cw[0, :, 0:1] + cw[1, :, 0:1]

    y2 = pl.pallas_call(
        _tc_mid_body,
        out_shape=jax.ShapeDtypeStruct((N_PAD, D), _f32),
    )(s1, y1, d_arr, b1.reshape(1, D), Wh)

    s2 = _spmm_kernel(y2, idx_fwd, zeros_t)

    out = pl.pallas_call(
        _tc_final_body,
        out_shape=jax.ShapeDtypeStruct((1, D), _f32),
    )(s2, y2, d_arr, bh.reshape(1, D), c_arr, Wo, bo.reshape(1, D))
    return out


# speedup vs baseline: 3.0461x; 3.0461x over previous
import functools

import jax
import jax.numpy as jnp
from jax import lax
from jax.experimental import pallas as pl
from jax.experimental.pallas import tpu as pltpu
from jax.experimental.pallas import tpu_sc as plsc

N_NODES = 10000
D = 128
N_EDGES = 320000

CHUNK = 128
CPW = 80
N_CHUNKS = 32 * CPW
E_PAD = N_CHUNKS * CHUNK
BLK = 8
N_BLK = CPW // BLK
N_PAD = 10240
ROWS_PER_TILE = N_PAD // 16

_mesh = plsc.VectorSubcoreMesh(core_axis_name="c", subcore_axis_name="s")
_f32 = jnp.float32
_sc_params = pltpu.CompilerParams(use_tc_tiling_on_sc=False)


@functools.partial(
    pl.kernel,
    out_type=jax.ShapeDtypeStruct((2, N_PAD, D), _f32),
    mesh=_mesh,
    scratch_types=[
        pltpu.VMEM((CHUNK,), jnp.int32),
        pltpu.VMEM((CHUNK, D), _f32),
        pltpu.VMEM((CHUNK, D), _f32),
        pltpu.VMEM_SHARED((N_PAD, D), _f32),
        pltpu.SemaphoreType.DMA,
    ],
    compiler_params=_sc_params,
)
def _deg_kernel(col_hbm, ones_hbm, zeros_hbm, deg_out, idx_v, ones_v, zero_v,
                acc_sh, sem):
    cid = lax.axis_index("c")
    sid = lax.axis_index("s")
    base = sid * ROWS_PER_TILE
    g0 = (sid * 2 + cid) * CPW

    pltpu.sync_copy(ones_hbm, ones_v)
    pltpu.sync_copy(zeros_hbm, zero_v)
    for k in range(ROWS_PER_TILE // CHUNK):
        pltpu.sync_copy(zero_v, acc_sh.at[pl.ds(base + k * CHUNK, CHUNK)])
    plsc.subcore_barrier()

    def body(j, carry):
        pltpu.sync_copy(col_hbm.at[g0 + j], idx_v)
        pltpu.sync_copy(ones_v, acc_sh.at[idx_v], add=True)
        return carry

    lax.fori_loop(0, CPW, body, 0)
    plsc.subcore_barrier()

    for k in range(ROWS_PER_TILE // CHUNK):
        pltpu.sync_copy(acc_sh.at[pl.ds(base + k * CHUNK, CHUNK)], zero_v)
        pltpu.sync_copy(zero_v, deg_out.at[cid, pl.ds(base + k * CHUNK, CHUNK)])


@functools.partial(
    pl.kernel,
    out_type=jax.ShapeDtypeStruct((2, N_PAD, D), _f32),
    mesh=_mesh,
    scratch_types=[
        pltpu.VMEM((4 * BLK, CHUNK), jnp.int32),
        pltpu.VMEM((CHUNK, D), _f32),
        pltpu.VMEM((CHUNK, D), _f32),
        pltpu.VMEM_SHARED((N_PAD, D), _f32),
        pltpu.SemaphoreType.DMA,
        pltpu.SemaphoreType.DMA,
        pltpu.SemaphoreType.DMA,
    ],
    compiler_params=_sc_params,
)
def _spmm_kernel(y_hbm, idx_hbm, zeros_hbm, s_out, idxp_v, gb_a, gb_b, acc_sh,
                 sem_a, sem_b, sem_i):
    cid = lax.axis_index("c")
    sid = lax.axis_index("s")
    base = sid * ROWS_PER_TILE
    r0_hbm = 2 * (sid * 2 + cid) * CPW
    gbs = (gb_a, gb_b)
    sems = (sem_a, sem_b)

    pltpu.sync_copy(zeros_hbm, gb_a)
    for k in range(ROWS_PER_TILE // CHUNK):
        pltpu.sync_copy(gb_a, acc_sh.at[pl.ds(base + k * CHUNK, CHUNK)])
    plsc.subcore_barrier()

    pltpu.sync_copy(idx_hbm.at[pl.ds(r0_hbm, 2 * BLK)],
                    idxp_v.at[pl.ds(0, 2 * BLK)])
    pltpu.async_copy(idx_hbm.at[pl.ds(r0_hbm + 2 * BLK, 2 * BLK)],
                     idxp_v.at[pl.ds(2 * BLK, 2 * BLK)], sem_i)
    pltpu.async_copy(y_hbm.at[idxp_v.at[0]], gb_a, sem_a)

    def block(b, carry):
        r0 = (b % 2) * (2 * BLK)
        rn = ((b + 1) % 2) * (2 * BLK)
        for i in range(BLK):
            cur, nxt = gbs[i % 2], gbs[(i + 1) % 2]
            scur, snxt = sems[i % 2], sems[(i + 1) % 2]
            if i < BLK - 1:
                pltpu.async_copy(y_hbm.at[idxp_v.at[r0 + 2 * i + 2]], nxt,
                                 snxt)
            else:
                @pl.when(b < N_BLK - 1)
                def _():
                    pltpu.make_async_copy(
                        idx_hbm.at[pl.ds(r0_hbm + (b + 1) * 2 * BLK, 2 * BLK)],
                        idxp_v.at[pl.ds(rn, 2 * BLK)], sem_i).wait()
                    pltpu.async_copy(y_hbm.at[idxp_v.at[rn]], nxt, snxt)
            pltpu.make_async_copy(y_hbm.at[idxp_v.at[r0 + 2 * i]], cur,
                                  scur).wait()
            pltpu.sync_copy(cur, acc_sh.at[idxp_v.at[r0 + 2 * i + 1]],
                            add=True)
            if i == BLK - 1:
                @pl.when(b < N_BLK - 2)
                def _():
                    pltpu.async_copy(
                        idx_hbm.at[pl.ds(r0_hbm + (b + 2) * 2 * BLK, 2 * BLK)],
                        idxp_v.at[pl.ds(r0, 2 * BLK)], sem_i)
        return carry

    lax.fori_loop(0, N_BLK, block, 0)
    plsc.subcore_barrier()

    for k in range(ROWS_PER_TILE // CHUNK):
        pltpu.sync_copy(acc_sh.at[pl.ds(base + k * CHUNK, CHUNK)], gb_a)
        pltpu.sync_copy(gb_a, s_out.at[cid, pl.ds(base + k * CHUNK, CHUNK)])


def _tc_prep_body(deg_ref, x_ref, w1_ref, d_ref, y_ref, dw_ref):
    d = lax.rsqrt(deg_ref[...] + 1.0)
    d_ref[...] = d
    xl = jnp.dot(x_ref[...], w1_ref[...], preferred_element_type=_f32)
    zpad = jnp.zeros((N_PAD - N_NODES, D), _f32)
    y_ref[0:N_NODES, :] = d[0:N_NODES] * xl
    y_ref[N_NODES:N_PAD, :] = zpad
    dw_ref[...] = jnp.broadcast_to(d, (N_PAD, D))


def _tc_mid_body(s_ref, y_ref, d_ref, b_ref, w_ref, out_ref):
    d = d_ref[0:N_NODES]
    s = s_ref[0, 0:N_NODES, :] + s_ref[1, 0:N_NODES, :] + y_ref[0:N_NODES, :]
    h = jnp.maximum(d * s + b_ref[...], 0.0)
    xl = jnp.dot(h, w_ref[...], preferred_element_type=_f32)
    out_ref[0:N_NODES, :] = d * xl
    out_ref[N_NODES:N_PAD, :] = jnp.zeros((N_PAD - N_NODES, D), _f32)


def _tc_final_body(s_ref, y_ref, d_ref, b_ref, c_ref, wo_ref, bo_ref,
                   out_ref):
    d = d_ref[0:N_NODES]
    s = s_ref[0, 0:N_NODES, :] + s_ref[1, 0:N_NODES, :] + y_ref[0:N_NODES, :]
    h2 = jnp.maximum(d * s + b_ref[...], 0.0)
    w = d * (c_ref[0:N_NODES] + d)
    z = jnp.sum(h2 * w, axis=0, keepdims=True)
    out_ref[...] = (jnp.dot(z, wo_ref[...], preferred_element_type=_f32)
                    * (1.0 / N_NODES) + bo_ref[...])


def kernel(x, edge_index, W1, b1, Wh, bh, Wo, bo):
    ei = edge_index.astype(jnp.int32)
    pad = N_NODES + jnp.arange(E_PAD - N_EDGES, dtype=jnp.int32) % (
        N_PAD - N_NODES)
    row2 = jnp.concatenate([ei[0], pad]).reshape(N_CHUNKS, CHUNK)
    col2 = jnp.concatenate([ei[1], pad]).reshape(N_CHUNKS, CHUNK)
    idx_fwd = jnp.stack([row2, col2], axis=1).reshape(2 * N_CHUNKS, CHUNK)
    idx_rev = jnp.stack([col2, row2], axis=1).reshape(2 * N_CHUNKS, CHUNK)

    ones_t = jnp.ones((CHUNK, D), _f32)
    zeros_t = jnp.zeros((CHUNK, D), _f32)

    degw = _deg_kernel(col2, ones_t, zeros_t)
    deg = degw[0, :, 0:1] + degw[1, :, 0:1]

    d_arr, y1, dwide = pl.pallas_call(
        _tc_prep_body,
        out_shape=(jax.ShapeDtypeStruct((N_PAD, 1), _f32),
                   jax.ShapeDtypeStruct((N_PAD, D), _f32),
                   jax.ShapeDtypeStruct((N_PAD, D), _f32)),
    )(deg, x, W1)

    s1 = _spmm_kernel(y1, idx_fwd, zeros_t)
    cw = _spmm_kernel(dwide, idx_rev, zeros_t)
    c_arr = cw[0, :, 0:1] + cw[1, :, 0:1]

    y2 = pl.pallas_call(
        _tc_mid_body,
        out_shape=jax.ShapeDtypeStruct((N_PAD, D), _f32),
    )(s1, y1, d_arr, b1.reshape(1, D), Wh)

    s2 = _spmm_kernel(y2, idx_fwd, zeros_t)

    out = pl.pallas_call(
        _tc_final_body,
        out_shape=jax.ShapeDtypeStruct((1, D), _f32),
    )(s2, y2, d_arr, bh.reshape(1, D), c_arr, Wo, bo.reshape(1, D))
    return out
